# Initial kernel scaffold; baseline (speedup 1.0000x reference)
import jax
import jax.numpy as jnp
from jax.experimental import pallas as pl


def kernel(logits, extended_vocab_ids, src_subtokens, src_padding, tgt_subtokens, len_vocab, max_len_extended_vocab, Wq, Wk, lin_w, lin_b):
    raise NotImplementedError("write your pallas kernel here")


# speedup vs baseline: 3.4266x; 3.4266x over previous
import functools
import math

import jax
import jax.numpy as jnp
from jax import lax
from jax.experimental import pallas as pl
from jax.experimental.pallas import tpu as pltpu
from jax.experimental.pallas import tpu_sc as plsc
import numpy as np

_B = 8
_SRC = 512
_TGT = 256
_D = 256
_H = 8
_LV = 8000
_MEXT = 512
_V = _LV + _MEXT
_DH = _D // _H

_EPS = float(np.finfo(np.float32).eps)
_FMIN = float(np.finfo(np.float32).min)
_NROWS = _B * _TGT


def _attn_body(ids_ref, tgt_ref, src_ref, wq_ref, wk_ref, lw_ref, lb_ref,
               a2_ref, scal_ref):
    tgt = tgt_ref[0]
    src = src_ref[0]
    lw = lw_ref[...]

    z = jnp.sum(tgt * lw, axis=1, keepdims=True) + lb_ref[...]
    cp = jax.nn.sigmoid(z)
    logcp = jnp.log(cp)
    log1m = jnp.log(1.0 - cp)
    scal_ref[0, :, 0:128] = jnp.broadcast_to(logcp, (_TGT, 128))
    scal_ref[0, :, 128:256] = jnp.broadcast_to(log1m, (_TGT, 128))

    q = jnp.dot(tgt, wq_ref[...], preferred_element_type=jnp.float32)
    k = jnp.dot(src, wk_ref[...], preferred_element_type=jnp.float32)
    inv = jnp.float32(1.0 / math.sqrt(_DH))
    acc = jnp.zeros((_TGT, _SRC), jnp.float32)
    for h in range(_H):
        qh = q[:, h * _DH:(h + 1) * _DH]
        kh = k[:, h * _DH:(h + 1) * _DH]
        sc = lax.dot_general(qh, kh, (((1,), (1,)), ((), ())),
                             preferred_element_type=jnp.float32) * inv
        m = jnp.max(sc, axis=1, keepdims=True)
        e = jnp.exp(sc - m)
        acc = acc + e / jnp.sum(e, axis=1, keepdims=True)
    attn = acc * jnp.float32(1.0 / _H)

    idf = ids_ref[0].astype(jnp.float32)
    row = jnp.broadcast_to(idf, (_SRC, _SRC))
    col = row.T
    eq = (row == col).astype(jnp.float32)
    a2_ref[0] = lax.dot_general(attn, eq, (((1,), (0,)), ((), ())),
                                preferred_element_type=jnp.float32)


def _attn_call(ids3, tgt, src, wq, wk, lw, lb):
    return pl.pallas_call(
        _attn_body,
        grid=(_B,),
        in_specs=[
            pl.BlockSpec((1, 1, _SRC), lambda b: (b, 0, 0)),
            pl.BlockSpec((1, _TGT, _D), lambda b: (b, 0, 0)),
            pl.BlockSpec((1, _SRC, _D), lambda b: (b, 0, 0)),
            pl.BlockSpec((_D, _D), lambda b: (0, 0)),
            pl.BlockSpec((_D, _D), lambda b: (0, 0)),
            pl.BlockSpec((1, _D), lambda b: (0, 0)),
            pl.BlockSpec((1, 1), lambda b: (0, 0)),
        ],
        out_specs=[
            pl.BlockSpec((1, _TGT, _SRC), lambda b: (b, 0, 0)),
            pl.BlockSpec((1, _TGT, 256), lambda b: (b, 0, 0)),
        ],
        out_shape=[
            jax.ShapeDtypeStruct((_B, _TGT, _SRC), jnp.float32),
            jax.ShapeDtypeStruct((_B, _TGT, 256), jnp.float32),
        ],
        compiler_params=pltpu.CompilerParams(
            dimension_semantics=("arbitrary",)),
    )(ids3, tgt, src, wq, wk, lw, lb)


_ROWS_PER_W = _NROWS // 32
_WPB = 4


@functools.cache
def _make_scatter_kernel():
    @functools.partial(
        pl.kernel,
        out_type=jax.ShapeDtypeStruct((_NROWS, _V), jnp.float32),
        mesh=plsc.VectorSubcoreMesh(core_axis_name="c", subcore_axis_name="s"),
        scratch_types=[
            pltpu.VMEM((_SRC,), jnp.int32),
            pltpu.VMEM((_SRC,), jnp.float32),
            pltpu.VMEM((_V,), jnp.float32),
        ],
        compiler_params=pltpu.CompilerParams(needs_layout_passes=False),
    )
    def _scatter_kernel(ids_hbm, a2_hbm, p_hbm, ids_v, a2_v, work_v):
        wid = lax.axis_index("s") * 2 + lax.axis_index("c")
        b = wid // _WPB
        base = b * _TGT + (wid % _WPB) * _ROWS_PER_W

        pltpu.sync_copy(ids_hbm.at[b], ids_v)

        zero = jnp.zeros((16,), jnp.float32)
        for i in range(_V // 16):
            work_v[pl.ds(i * 16, 16)] = zero

        def row_body(j, carry):
            r = base + j
            pltpu.sync_copy(a2_hbm.at[r], a2_v)
            for c in range(_SRC // 16):
                iv = ids_v[pl.ds(c * 16, 16)]
                vv = a2_v[pl.ds(c * 16, 16)]
                plsc.store_scatter(work_v, [iv], vv)
            pltpu.sync_copy(work_v, p_hbm.at[r])
            return carry

        lax.fori_loop(0, _ROWS_PER_W, row_body, 0)

    return _scatter_kernel


def _lse_body(x_ref, lse_ref):
    x = x_ref[0]
    m = jnp.max(x, axis=1, keepdims=True)
    lse = m + jnp.log(jnp.sum(jnp.exp(x - m), axis=1, keepdims=True))
    lse_ref[0] = jnp.broadcast_to(lse, (_TGT, 128))


def _lse_call(logits):
    return pl.pallas_call(
        _lse_body,
        grid=(_B,),
        in_specs=[pl.BlockSpec((1, _TGT, _LV), lambda b: (b, 0, 0))],
        out_specs=pl.BlockSpec((1, _TGT, 128), lambda b: (b, 0, 0)),
        out_shape=jax.ShapeDtypeStruct((_B, _TGT, 128), jnp.float32),
        compiler_params=pltpu.CompilerParams(
            dimension_semantics=("arbitrary",)),
    )(logits)


_VB = 1152
_NVB = -(-_V // _VB)
_NXB = -(-_LV // _VB)


def _combine_body(x_ref, p_ref, scal_ref, lse_ref, out_ref):
    vb = pl.program_id(1)
    logcp = scal_ref[0, :, 0:1]
    log1m = scal_ref[0, :, 128:129]
    lse = lse_ref[0, :, 0:1]

    x = x_ref[0]
    vidx = lax.broadcasted_iota(jnp.int32, (_TGT, _VB), 1) + vb * _VB
    p0 = x - lse + log1m
    p0 = jnp.where(p0 == -jnp.inf, _FMIN, p0)
    p0 = jnp.where(vidx < _LV, p0, _FMIN)

    pm = p_ref[0]
    p1 = jnp.log(jnp.where(pm == 0.0, _EPS, pm)) + logcp
    p1 = jnp.where(p1 == -jnp.inf, _FMIN, p1)

    mx = jnp.maximum(p0, p1)
    out_ref[0] = mx + jnp.log1p(jnp.exp(-jnp.abs(p0 - p1)))


def _combine_call(logits, p, scal, lse):
    return pl.pallas_call(
        _combine_body,
        grid=(_B, _NVB),
        in_specs=[
            pl.BlockSpec((1, _TGT, _VB),
                         lambda b, v: (b, 0, jnp.minimum(v, _NXB - 1))),
            pl.BlockSpec((1, _TGT, _VB), lambda b, v: (b, 0, v)),
            pl.BlockSpec((1, _TGT, 256), lambda b, v: (b, 0, 0)),
            pl.BlockSpec((1, _TGT, 128), lambda b, v: (b, 0, 0)),
        ],
        out_specs=pl.BlockSpec((1, _TGT, _VB), lambda b, v: (b, 0, v)),
        out_shape=jax.ShapeDtypeStruct((_B, _TGT, _V), jnp.float32),
        compiler_params=pltpu.CompilerParams(
            dimension_semantics=("arbitrary", "arbitrary")),
    )(logits, p, scal, lse)


def kernel(logits, extended_vocab_ids, src_subtokens, src_padding,
           tgt_subtokens, len_vocab, max_len_extended_vocab,
           Wq, Wk, lin_w, lin_b):
    ids3 = extended_vocab_ids.reshape(_B, 1, _SRC)
    lw = lin_w.reshape(1, _D)
    lb = lin_b.reshape(1, 1)

    a2, scal = _attn_call(ids3, tgt_subtokens, src_subtokens, Wq, Wk, lw, lb)
    p = _make_scatter_kernel()(extended_vocab_ids, a2.reshape(_NROWS, _SRC))
    lse = _lse_call(logits)
    out = _combine_call(logits, p.reshape(_B, _TGT, _V), scal, lse)
    return out


# speedup vs baseline: 3.5593x; 1.0387x over previous
import functools
import math

import jax
import jax.numpy as jnp
from jax import lax
from jax.experimental import pallas as pl
from jax.experimental.pallas import tpu as pltpu
from jax.experimental.pallas import tpu_sc as plsc
import numpy as np

_B = 8
_SRC = 512
_TGT = 256
_D = 256
_H = 8
_LV = 8000
_MEXT = 512
_V = _LV + _MEXT
_DH = _D // _H

_EPS = float(np.finfo(np.float32).eps)
_FMIN = float(np.finfo(np.float32).min)
_NROWS = _B * _TGT


def _attn_body(ids_ref, tgt_ref, src_ref, wq_ref, wk_ref, lw_ref, lb_ref,
               a2_ref, scal_ref):
    tgt = tgt_ref[0]
    src = src_ref[0]
    lw = lw_ref[...]

    z = jnp.sum(tgt * lw, axis=1, keepdims=True) + lb_ref[...]
    cp = jax.nn.sigmoid(z)
    logcp = jnp.log(cp)
    log1m = jnp.log(1.0 - cp)
    scal_ref[0, :, 0:128] = jnp.broadcast_to(logcp, (_TGT, 128))
    scal_ref[0, :, 128:256] = jnp.broadcast_to(log1m, (_TGT, 128))

    q = jnp.dot(tgt, wq_ref[...], preferred_element_type=jnp.float32)
    k = jnp.dot(src, wk_ref[...], preferred_element_type=jnp.float32)
    inv = jnp.float32(1.0 / math.sqrt(_DH))
    acc = jnp.zeros((_TGT, _SRC), jnp.float32)
    for h in range(_H):
        qh = q[:, h * _DH:(h + 1) * _DH]
        kh = k[:, h * _DH:(h + 1) * _DH]
        sc = lax.dot_general(qh, kh, (((1,), (1,)), ((), ())),
                             preferred_element_type=jnp.float32) * inv
        m = jnp.max(sc, axis=1, keepdims=True)
        e = jnp.exp(sc - m)
        acc = acc + e / jnp.sum(e, axis=1, keepdims=True)
    attn = acc * jnp.float32(1.0 / _H)

    idf = ids_ref[0].astype(jnp.float32)
    row = jnp.broadcast_to(idf, (_SRC, _SRC))
    col = row.T
    eq = (row == col).astype(jnp.float32)
    a2_ref[0] = lax.dot_general(attn, eq, (((1,), (0,)), ((), ())),
                                preferred_element_type=jnp.float32)


def _attn_call(ids3, tgt, src, wq, wk, lw, lb):
    return pl.pallas_call(
        _attn_body,
        grid=(_B,),
        in_specs=[
            pl.BlockSpec((1, 1, _SRC), lambda b: (b, 0, 0)),
            pl.BlockSpec((1, _TGT, _D), lambda b: (b, 0, 0)),
            pl.BlockSpec((1, _SRC, _D), lambda b: (b, 0, 0)),
            pl.BlockSpec((_D, _D), lambda b: (0, 0)),
            pl.BlockSpec((_D, _D), lambda b: (0, 0)),
            pl.BlockSpec((1, _D), lambda b: (0, 0)),
            pl.BlockSpec((1, 1), lambda b: (0, 0)),
        ],
        out_specs=[
            pl.BlockSpec((1, _TGT, _SRC), lambda b: (b, 0, 0)),
            pl.BlockSpec((1, _TGT, 256), lambda b: (b, 0, 0)),
        ],
        out_shape=[
            jax.ShapeDtypeStruct((_B, _TGT, _SRC), jnp.float32),
            jax.ShapeDtypeStruct((_B, _TGT, 256), jnp.float32),
        ],
        compiler_params=pltpu.CompilerParams(
            dimension_semantics=("arbitrary",)),
    )(ids3, tgt, src, wq, wk, lw, lb)


_ROWS_PER_W = _NROWS // 32
_WPB = 4


@functools.cache
def _make_scatter_kernel():
    @functools.partial(
        pl.kernel,
        out_type=jax.ShapeDtypeStruct((_NROWS, _V), jnp.float32),
        mesh=plsc.VectorSubcoreMesh(core_axis_name="c", subcore_axis_name="s"),
        scratch_types=[
            pltpu.VMEM((_SRC,), jnp.int32),
            pltpu.VMEM((_SRC,), jnp.float32),
            pltpu.VMEM((_SRC,), jnp.float32),
            pltpu.VMEM((_V,), jnp.float32),
            pltpu.VMEM((_V,), jnp.float32),
            pltpu.SemaphoreType.DMA,
            pltpu.SemaphoreType.DMA,
            pltpu.SemaphoreType.DMA,
            pltpu.SemaphoreType.DMA,
        ],
        compiler_params=pltpu.CompilerParams(needs_layout_passes=False),
    )
    def _scatter_kernel(ids_hbm, a2_hbm, p_hbm, ids_v, a2_v0, a2_v1,
                        work_v0, work_v1, sem_a0, sem_a1, sem_w0, sem_w1):
        wid = lax.axis_index("s") * 2 + lax.axis_index("c")
        b = wid // _WPB
        base = b * _TGT + (wid % _WPB) * _ROWS_PER_W

        pltpu.sync_copy(ids_hbm.at[b], ids_v)

        zero = jnp.zeros((16,), jnp.float32)
        for i in range(_V // 16):
            work_v0[pl.ds(i * 16, 16)] = zero
            work_v1[pl.ds(i * 16, 16)] = zero

        pltpu.async_copy(a2_hbm.at[base], a2_v0, sem_a0)
        pltpu.async_copy(a2_hbm.at[base + 1], a2_v1, sem_a1)

        def _half(i, r, a2_v, work_v, sem_a, sem_w):
            pltpu.make_async_copy(a2_hbm.at[r], a2_v, sem_a).wait()

            @pl.when(i > 0)
            def _():
                pltpu.make_async_copy(work_v, p_hbm.at[r], sem_w).wait()

            for c in range(_SRC // 16):
                iv = ids_v[pl.ds(c * 16, 16)]
                vv = a2_v[pl.ds(c * 16, 16)]
                plsc.store_scatter(work_v, [iv], vv)
            pltpu.async_copy(work_v, p_hbm.at[r], sem_w)

            @pl.when(r + 2 < base + _ROWS_PER_W)
            def _():
                pltpu.async_copy(a2_hbm.at[r + 2], a2_v, sem_a)

        def row_body(i, carry):
            _half(i, base + 2 * i, a2_v0, work_v0, sem_a0, sem_w0)
            _half(i, base + 2 * i + 1, a2_v1, work_v1, sem_a1, sem_w1)
            return carry

        lax.fori_loop(0, _ROWS_PER_W // 2, row_body, 0)
        last = base + _ROWS_PER_W
        pltpu.make_async_copy(work_v0, p_hbm.at[last - 2], sem_w0).wait()
        pltpu.make_async_copy(work_v1, p_hbm.at[last - 1], sem_w1).wait()

    return _scatter_kernel


_VB = 1152
_NVB = -(-_V // _VB)
_NXB = -(-_LV // _VB)
_XW = _NXB * _VB


def _combine_body(x_ref, p_ref, scal_ref, out_ref, lse_ref):
    vb = pl.program_id(1)
    logcp = scal_ref[0, :, 0:1]
    log1m = scal_ref[0, :, 128:129]

    @pl.when(vb == 0)
    def _():
        xf = x_ref[0]
        cols = lax.broadcasted_iota(jnp.int32, (_TGT, _XW), 1)
        xm = jnp.where(cols < _LV, xf, _FMIN)
        m = jnp.max(xm, axis=1, keepdims=True)
        lse = m + jnp.log(jnp.sum(jnp.exp(xm - m), axis=1, keepdims=True))
        lse_ref[...] = jnp.broadcast_to(lse, (_TGT, 128))

    lse = lse_ref[:, 0:1]
    xstart = jnp.minimum(vb, _NXB - 1) * _VB
    x = x_ref[0, :, pl.ds(xstart, _VB)]
    vidx = lax.broadcasted_iota(jnp.int32, (_TGT, _VB), 1) + vb * _VB
    p0 = x - lse + log1m
    p0 = jnp.where(p0 == -jnp.inf, _FMIN, p0)
    p0 = jnp.where(vidx < _LV, p0, _FMIN)

    pm = p_ref[0]
    p1 = jnp.log(jnp.where(pm == 0.0, _EPS, pm)) + logcp
    p1 = jnp.where(p1 == -jnp.inf, _FMIN, p1)

    mx = jnp.maximum(p0, p1)
    out_ref[0] = mx + jnp.log1p(jnp.exp(-jnp.abs(p0 - p1)))


def _combine_call(logits, p, scal):
    return pl.pallas_call(
        _combine_body,
        grid=(_B, _NVB),
        in_specs=[
            pl.BlockSpec((1, _TGT, _XW), lambda b, v: (b, 0, 0)),
            pl.BlockSpec((1, _TGT, _VB), lambda b, v: (b, 0, v)),
            pl.BlockSpec((1, _TGT, 256), lambda b, v: (b, 0, 0)),
        ],
        out_specs=pl.BlockSpec((1, _TGT, _VB), lambda b, v: (b, 0, v)),
        out_shape=jax.ShapeDtypeStruct((_B, _TGT, _V), jnp.float32),
        scratch_shapes=[pltpu.VMEM((_TGT, 128), jnp.float32)],
        compiler_params=pltpu.CompilerParams(
            dimension_semantics=("arbitrary", "arbitrary")),
    )(logits, p, scal)


def kernel(logits, extended_vocab_ids, src_subtokens, src_padding,
           tgt_subtokens, len_vocab, max_len_extended_vocab,
           Wq, Wk, lin_w, lin_b):
    ids3 = extended_vocab_ids.reshape(_B, 1, _SRC)
    lw = lin_w.reshape(1, _D)
    lb = lin_b.reshape(1, 1)

    a2, scal = _attn_call(ids3, tgt_subtokens, src_subtokens, Wq, Wk, lw, lb)
    p = _make_scatter_kernel()(extended_vocab_ids, a2.reshape(_NROWS, _SRC))
    out = _combine_call(logits, p.reshape(_B, _TGT, _V), scal)
    return out


# speedup vs baseline: 5.7061x; 1.6032x over previous
import functools
import math

import jax
import jax.numpy as jnp
from jax import lax
from jax.experimental import pallas as pl
from jax.experimental.pallas import tpu as pltpu
from jax.experimental.pallas import tpu_sc as plsc
import numpy as np

_B = 8
_SRC = 512
_TGT = 256
_D = 256
_H = 8
_LV = 8000
_MEXT = 512
_V = _LV + _MEXT
_DH = _D // _H

_EPS = float(np.finfo(np.float32).eps)
_FMIN = float(np.finfo(np.float32).min)

_VB = 1152
_NVB = -(-_V // _VB)
_NXB = -(-_LV // _VB)
_XW = _NXB * _VB


def _attn_body(ids_ref, tgt_ref, src_ref, wq_ref, wk_ref, lw_ref, lb_ref,
               a2t_ref, scal_ref):
    tgt = tgt_ref[0]
    src = src_ref[0]
    lw = lw_ref[...]

    z = lax.dot_general(lw, tgt, (((1,), (1,)), ((), ())),
                        preferred_element_type=jnp.float32)
    cp = jax.nn.sigmoid(z + lb_ref[...])
    scal_ref[0, 0:1, :] = jnp.log(cp)
    scal_ref[0, 1:2, :] = jnp.log(1.0 - cp)

    q = jnp.dot(tgt, wq_ref[...], preferred_element_type=jnp.float32)
    k = jnp.dot(src, wk_ref[...], preferred_element_type=jnp.float32)
    inv = jnp.float32(1.0 / math.sqrt(_DH))
    acc = jnp.zeros((_TGT, _SRC), jnp.float32)
    for h in range(_H):
        qh = q[:, h * _DH:(h + 1) * _DH]
        kh = k[:, h * _DH:(h + 1) * _DH]
        sc = lax.dot_general(qh, kh, (((1,), (1,)), ((), ())),
                             preferred_element_type=jnp.float32) * inv
        m = jnp.max(sc, axis=1, keepdims=True)
        e = jnp.exp(sc - m)
        acc = acc + e / jnp.sum(e, axis=1, keepdims=True)
    attn = acc * jnp.float32(1.0 / _H)

    idf = ids_ref[0].astype(jnp.float32)
    row = jnp.broadcast_to(idf, (_SRC, _SRC))
    col = row.T
    eq = (row == col).astype(jnp.float32)
    a2t_ref[0] = lax.dot_general(eq, attn, (((1,), (1,)), ((), ())),
                                 preferred_element_type=jnp.float32)


def _attn_call(ids3, tgt, src, wq, wk, lw, lb):
    return pl.pallas_call(
        _attn_body,
        grid=(_B,),
        in_specs=[
            pl.BlockSpec((1, 1, _SRC), lambda b: (b, 0, 0)),
            pl.BlockSpec((1, _TGT, _D), lambda b: (b, 0, 0)),
            pl.BlockSpec((1, _SRC, _D), lambda b: (b, 0, 0)),
            pl.BlockSpec((_D, _D), lambda b: (0, 0)),
            pl.BlockSpec((_D, _D), lambda b: (0, 0)),
            pl.BlockSpec((1, _D), lambda b: (0, 0)),
            pl.BlockSpec((1, 1), lambda b: (0, 0)),
        ],
        out_specs=[
            pl.BlockSpec((1, _SRC, _TGT), lambda b: (b, 0, 0)),
            pl.BlockSpec((1, 8, _TGT), lambda b: (b, 0, 0)),
        ],
        out_shape=[
            jax.ShapeDtypeStruct((_B, _SRC, _TGT), jnp.float32),
            jax.ShapeDtypeStruct((_B, 8, _TGT), jnp.float32),
        ],
        compiler_params=pltpu.CompilerParams(
            dimension_semantics=("arbitrary",)),
    )(ids3, tgt, src, wq, wk, lw, lb)


_WPB = 4
_VSLAB = _V // _WPB
_ZR = 16
_NZ = _VSLAB // _ZR
_SCHUNK = _SRC // _WPB


@functools.cache
def _make_scatter_kernel():
    @functools.partial(
        pl.kernel,
        out_type=jax.ShapeDtypeStruct((_B * _V, _TGT), jnp.float32),
        mesh=plsc.VectorSubcoreMesh(core_axis_name="c", subcore_axis_name="s"),
        scratch_types=[
            pltpu.VMEM((_ZR, _TGT), jnp.float32),
            pltpu.VMEM((_SCHUNK, _TGT), jnp.float32),
            pltpu.VMEM((1, _SCHUNK), jnp.int32),
            pltpu.VMEM((1, _SCHUNK), jnp.int32),
            pltpu.SemaphoreType.DMA,
            pltpu.SemaphoreType.DMA,
        ],
        compiler_params=pltpu.CompilerParams(needs_layout_passes=False),
    )
    def _scatter_kernel(ids_hbm, a2t_hbm, pt_hbm, zbuf, a2buf, idsbuf,
                        idxbuf, sem_z, sem_s):
        core = lax.axis_index("c")
        sub = lax.axis_index("s")
        b = core * 4 + sub // _WPB
        w4 = sub % _WPB

        zero = jnp.zeros((16,), jnp.float32)
        for r in range(_ZR):
            for c in range(_TGT // 16):
                zbuf[r, pl.ds(c * 16, 16)] = zero

        row0 = b * _V + w4 * _VSLAB

        def zfill(i, carry):
            pltpu.async_copy(
                zbuf, pt_hbm.at[pl.ds(row0 + i * _ZR, _ZR)], sem_z)
            return carry

        lax.fori_loop(0, _NZ, zfill, 0)

        s0 = w4 * _SCHUNK
        pltpu.sync_copy(a2t_hbm.at[b, pl.ds(s0, _SCHUNK)], a2buf)
        pltpu.sync_copy(ids_hbm.at[b, pl.ds(s0, _SCHUNK)], idsbuf.at[0])
        for c in range(_SCHUNK // 16):
            idxbuf[0, pl.ds(c * 16, 16)] = (
                idsbuf[0, pl.ds(c * 16, 16)] + b * _V)

        def zdrain(i, carry):
            pltpu.make_async_copy(
                zbuf, pt_hbm.at[pl.ds(row0, _ZR)], sem_z).wait()
            return carry

        lax.fori_loop(0, _NZ, zdrain, 0)
        plsc.subcore_barrier()
        pltpu.async_copy(a2buf, pt_hbm.at[idxbuf.at[0]], sem_s)
        pltpu.make_async_copy(a2buf, pt_hbm.at[idxbuf.at[0]], sem_s).wait()

    return _scatter_kernel


def _combine_body(x_ref, p_ref, scal_ref, out_ref, lse_ref):
    vb = pl.program_id(1)
    logcp = scal_ref[0, 0:1, :]
    log1m = scal_ref[0, 1:2, :]

    @pl.when(vb == 0)
    def _():
        m = jnp.full((1, _TGT), _FMIN, jnp.float32)
        for k in range(_NXB):
            xk = x_ref[0, k * _VB:(k + 1) * _VB, :]
            if (k + 1) * _VB > _LV:
                rows = lax.broadcasted_iota(jnp.int32, (_VB, _TGT), 0)
                xk = jnp.where(rows < _LV - k * _VB, xk, _FMIN)
            m = jnp.maximum(m, jnp.max(xk, axis=0, keepdims=True))
        s = jnp.zeros((1, _TGT), jnp.float32)
        for k in range(_NXB):
            xk = x_ref[0, k * _VB:(k + 1) * _VB, :]
            if (k + 1) * _VB > _LV:
                rows = lax.broadcasted_iota(jnp.int32, (_VB, _TGT), 0)
                xk = jnp.where(rows < _LV - k * _VB, xk, _FMIN)
            s = s + jnp.sum(jnp.exp(xk - m), axis=0, keepdims=True)
        lse_ref[...] = jnp.broadcast_to(m + jnp.log(s), (8, _TGT))

    lse = lse_ref[0:1, :]
    xstart = jnp.minimum(vb, _NXB - 1) * _VB
    x = x_ref[0, pl.ds(xstart, _VB), :]
    vidx = lax.broadcasted_iota(jnp.int32, (_VB, _TGT), 0) + vb * _VB
    p0 = x - lse + log1m
    p0 = jnp.where(p0 == -jnp.inf, _FMIN, p0)
    p0 = jnp.where(vidx < _LV, p0, _FMIN)

    pm = p_ref[0]
    p1 = jnp.log(jnp.where(pm == 0.0, _EPS, pm)) + logcp
    p1 = jnp.where(p1 == -jnp.inf, _FMIN, p1)

    mx = jnp.maximum(p0, p1)
    out_ref[0] = mx + jnp.log1p(jnp.exp(-jnp.abs(p0 - p1)))


def _combine_call(logits_t, pt, scal):
    return pl.pallas_call(
        _combine_body,
        grid=(_B, _NVB),
        in_specs=[
            pl.BlockSpec((1, _XW, _TGT), lambda b, v: (b, 0, 0)),
            pl.BlockSpec((1, _VB, _TGT), lambda b, v: (b, v, 0)),
            pl.BlockSpec((1, 8, _TGT), lambda b, v: (b, 0, 0)),
        ],
        out_specs=pl.BlockSpec((1, _VB, _TGT), lambda b, v: (b, v, 0)),
        out_shape=jax.ShapeDtypeStruct((_B, _V, _TGT), jnp.float32),
        scratch_shapes=[pltpu.VMEM((8, _TGT), jnp.float32)],
        compiler_params=pltpu.CompilerParams(
            dimension_semantics=("arbitrary", "arbitrary")),
    )(logits_t, pt, scal)


def kernel(logits, extended_vocab_ids, src_subtokens, src_padding,
           tgt_subtokens, len_vocab, max_len_extended_vocab,
           Wq, Wk, lin_w, lin_b):
    ids3 = extended_vocab_ids.reshape(_B, 1, _SRC)
    lw = lin_w.reshape(1, _D)
    lb = lin_b.reshape(1, 1)
    logits_t = jnp.swapaxes(logits, 1, 2)

    a2t, scal = _attn_call(ids3, tgt_subtokens, src_subtokens, Wq, Wk, lw, lb)
    pt = _make_scatter_kernel()(extended_vocab_ids, a2t)
    out_t = _combine_call(logits_t, pt.reshape(_B, _V, _TGT), scal)
    return jnp.swapaxes(out_t, 1, 2)


# speedup vs baseline: 7.1989x; 1.2616x over previous
import functools
import math

import jax
import jax.numpy as jnp
from jax import lax
from jax.experimental import pallas as pl
from jax.experimental.pallas import tpu as pltpu
from jax.experimental.pallas import tpu_sc as plsc
import numpy as np

_B = 8
_SRC = 512
_TGT = 256
_D = 256
_H = 8
_LV = 8000
_MEXT = 512
_V = _LV + _MEXT
_DH = _D // _H

_EPS = float(np.finfo(np.float32).eps)
_LOG_EPS = float(np.log(np.finfo(np.float32).eps))
_FMIN = float(np.finfo(np.float32).min)

_VB = 1152
_NVB = -(-_V // _VB)
_NXB = -(-_LV // _VB)
_XW = _NXB * _VB

_WPB = 4
_SCHUNK = _SRC // _WPB


@functools.cache
def _make_gather_kernel():
    @functools.partial(
        pl.kernel,
        out_type=jax.ShapeDtypeStruct((_B * _SRC, _TGT), jnp.float32),
        mesh=plsc.VectorSubcoreMesh(core_axis_name="c", subcore_axis_name="s"),
        scratch_types=[
            pltpu.VMEM((_SCHUNK, _TGT), jnp.float32),
            pltpu.VMEM((1, _SCHUNK), jnp.int32),
            pltpu.VMEM((1, _SCHUNK), jnp.int32),
            pltpu.SemaphoreType.DMA,
        ],
        compiler_params=pltpu.CompilerParams(needs_layout_passes=False),
    )
    def _gather_kernel(ids_hbm, xt_hbm, g_hbm, gbuf, idsbuf, idxbuf, sem):
        core = lax.axis_index("c")
        sub = lax.axis_index("s")
        b = core * 4 + sub // _WPB
        w4 = sub % _WPB
        s0 = w4 * _SCHUNK

        pltpu.sync_copy(ids_hbm.at[b, pl.ds(s0, _SCHUNK)], idsbuf.at[0])
        for c in range(_SCHUNK // 16):
            iv = idsbuf[0, pl.ds(c * 16, 16)]
            idxbuf[0, pl.ds(c * 16, 16)] = (
                jnp.minimum(iv, _LV - 1) + b * _LV)
        pltpu.async_copy(xt_hbm.at[idxbuf.at[0]], gbuf, sem)
        pltpu.make_async_copy(xt_hbm.at[idxbuf.at[0]], gbuf, sem).wait()
        pltpu.sync_copy(gbuf, g_hbm.at[pl.ds(b * _SRC + s0, _SCHUNK)])

    return _gather_kernel


def _attn_body(ids_ref, tgt_ref, src_ref, wq_ref, wk_ref, lw_ref, lb_ref,
               a2t_ref, scal_ref, idc_ref):
    tgt = tgt_ref[0]
    src = src_ref[0]
    lw = lw_ref[...]

    z = lax.dot_general(lw, tgt, (((1,), (1,)), ((), ())),
                        preferred_element_type=jnp.float32)
    cp = jax.nn.sigmoid(z + lb_ref[...])
    scal_ref[0, 0:1, :] = jnp.log(cp)
    scal_ref[0, 1:2, :] = jnp.log(1.0 - cp)

    q = jnp.dot(tgt, wq_ref[...], preferred_element_type=jnp.float32)
    k = jnp.dot(src, wk_ref[...], preferred_element_type=jnp.float32)
    inv = jnp.float32(1.0 / math.sqrt(_DH))
    acc = jnp.zeros((_TGT, _SRC), jnp.float32)
    for h in range(_H):
        qh = q[:, h * _DH:(h + 1) * _DH]
        kh = k[:, h * _DH:(h + 1) * _DH]
        sc = lax.dot_general(qh, kh, (((1,), (1,)), ((), ())),
                             preferred_element_type=jnp.float32) * inv
        m = jnp.max(sc, axis=1, keepdims=True)
        e = jnp.exp(sc - m)
        acc = acc + e / jnp.sum(e, axis=1, keepdims=True)
    attn = acc * jnp.float32(1.0 / _H)

    idf = ids_ref[0].astype(jnp.float32)
    row = jnp.broadcast_to(idf, (_SRC, _SRC))
    col = row.T
    idc_ref[0] = col[:, 0:128]
    eq = (row == col).astype(jnp.float32)
    a2t_ref[0] = lax.dot_general(eq, attn, (((1,), (1,)), ((), ())),
                                 preferred_element_type=jnp.float32)


def _attn_call(ids3, tgt, src, wq, wk, lw, lb):
    return pl.pallas_call(
        _attn_body,
        grid=(_B,),
        in_specs=[
            pl.BlockSpec((1, 1, _SRC), lambda b: (b, 0, 0)),
            pl.BlockSpec((1, _TGT, _D), lambda b: (b, 0, 0)),
            pl.BlockSpec((1, _SRC, _D), lambda b: (b, 0, 0)),
            pl.BlockSpec((_D, _D), lambda b: (0, 0)),
            pl.BlockSpec((_D, _D), lambda b: (0, 0)),
            pl.BlockSpec((1, _D), lambda b: (0, 0)),
            pl.BlockSpec((1, 1), lambda b: (0, 0)),
        ],
        out_specs=[
            pl.BlockSpec((1, _SRC, _TGT), lambda b: (b, 0, 0)),
            pl.BlockSpec((1, 8, _TGT), lambda b: (b, 0, 0)),
            pl.BlockSpec((1, _SRC, 128), lambda b: (b, 0, 0)),
        ],
        out_shape=[
            jax.ShapeDtypeStruct((_B, _SRC, _TGT), jnp.float32),
            jax.ShapeDtypeStruct((_B, 8, _TGT), jnp.float32),
            jax.ShapeDtypeStruct((_B, _SRC, 128), jnp.float32),
        ],
        compiler_params=pltpu.CompilerParams(
            dimension_semantics=("arbitrary",)),
    )(ids3, tgt, src, wq, wk, lw, lb)


def _combine_body(x_ref, scal_ref, g_ref, a2t_ref, idc_ref,
                  out_ref, u_ref, lse_ref):
    vb = pl.program_id(1)
    logcp = scal_ref[0, 0:1, :]
    log1m = scal_ref[0, 1:2, :]

    @pl.when(vb == 0)
    def _():
        m = jnp.full((1, _TGT), _FMIN, jnp.float32)
        for k in range(_NXB):
            xk = x_ref[0, k * _VB:(k + 1) * _VB, :]
            if (k + 1) * _VB > _LV:
                rows = lax.broadcasted_iota(jnp.int32, (_VB, _TGT), 0)
                xk = jnp.where(rows < _LV - k * _VB, xk, _FMIN)
            m = jnp.maximum(m, jnp.max(xk, axis=0, keepdims=True))
        s = jnp.zeros((1, _TGT), jnp.float32)
        for k in range(_NXB):
            xk = x_ref[0, k * _VB:(k + 1) * _VB, :]
            if (k + 1) * _VB > _LV:
                rows = lax.broadcasted_iota(jnp.int32, (_VB, _TGT), 0)
                xk = jnp.where(rows < _LV - k * _VB, xk, _FMIN)
            s = s + jnp.sum(jnp.exp(xk - m), axis=0, keepdims=True)
        lse = m + jnp.log(s)
        lse_ref[...] = jnp.broadcast_to(lse, (8, _TGT))

        gd = jnp.where(idc_ref[0, :, 0:1] < float(_LV),
                       g_ref[0] - lse, -jnp.inf)
        p0u = gd + log1m
        p0u = jnp.where(p0u == -jnp.inf, _FMIN, p0u)
        a2 = a2t_ref[0]
        p1u = jnp.log(jnp.where(a2 == 0.0, _EPS, a2)) + logcp
        p1u = jnp.where(p1u == -jnp.inf, _FMIN, p1u)
        mu = jnp.maximum(p0u, p1u)
        u_ref[0] = mu + jnp.log1p(jnp.exp(-jnp.abs(p0u - p1u)))

    lse = lse_ref[0:1, :]
    xstart = jnp.minimum(vb, _NXB - 1) * _VB
    x = x_ref[0, pl.ds(xstart, _VB), :]
    vidx = lax.broadcasted_iota(jnp.int32, (_VB, _TGT), 0) + vb * _VB
    p0 = x - lse + log1m
    p0 = jnp.where(p0 == -jnp.inf, _FMIN, p0)
    p0 = jnp.where(vidx < _LV, p0, _FMIN)

    p1 = logcp + _LOG_EPS
    p1 = jnp.where(p1 == -jnp.inf, _FMIN, p1)
    p1 = jnp.broadcast_to(p1, (_VB, _TGT))

    mx = jnp.maximum(p0, p1)
    out_ref[0] = mx + jnp.log1p(jnp.exp(-jnp.abs(p0 - p1)))


def _combine_call(logits_t, scal, g, a2t, idc):
    return pl.pallas_call(
        _combine_body,
        grid=(_B, _NVB),
        in_specs=[
            pl.BlockSpec((1, _XW, _TGT), lambda b, v: (b, 0, 0)),
            pl.BlockSpec((1, 8, _TGT), lambda b, v: (b, 0, 0)),
            pl.BlockSpec((1, _SRC, _TGT), lambda b, v: (b, 0, 0)),
            pl.BlockSpec((1, _SRC, _TGT), lambda b, v: (b, 0, 0)),
            pl.BlockSpec((1, _SRC, 128), lambda b, v: (b, 0, 0)),
        ],
        out_specs=[
            pl.BlockSpec((1, _VB, _TGT), lambda b, v: (b, v, 0)),
            pl.BlockSpec((1, _SRC, _TGT), lambda b, v: (b, 0, 0)),
        ],
        out_shape=[
            jax.ShapeDtypeStruct((_B, _V, _TGT), jnp.float32),
            jax.ShapeDtypeStruct((_B, _SRC, _TGT), jnp.float32),
        ],
        scratch_shapes=[pltpu.VMEM((8, _TGT), jnp.float32)],
        compiler_params=pltpu.CompilerParams(
            dimension_semantics=("arbitrary", "arbitrary")),
    )(logits_t, scal, g, a2t, idc)


@functools.cache
def _make_finalize_kernel():
    @functools.partial(
        pl.kernel,
        out_type=(),
        mesh=plsc.VectorSubcoreMesh(core_axis_name="c", subcore_axis_name="s"),
        scratch_types=[
            pltpu.VMEM((_SCHUNK, _TGT), jnp.float32),
            pltpu.VMEM((1, _SCHUNK), jnp.int32),
            pltpu.VMEM((1, _SCHUNK), jnp.int32),
            pltpu.SemaphoreType.DMA,
        ],
        compiler_params=pltpu.CompilerParams(needs_layout_passes=False),
    )
    def _finalize_kernel(ids_hbm, u_hbm, out_hbm, ubuf, idsbuf, idxbuf, sem):
        core = lax.axis_index("c")
        sub = lax.axis_index("s")
        b = core * 4 + sub // _WPB
        w4 = sub % _WPB
        s0 = w4 * _SCHUNK

        pltpu.sync_copy(ids_hbm.at[b, pl.ds(s0, _SCHUNK)], idsbuf.at[0])
        for c in range(_SCHUNK // 16):
            idxbuf[0, pl.ds(c * 16, 16)] = (
                idsbuf[0, pl.ds(c * 16, 16)] + b * _V)
        pltpu.sync_copy(u_hbm.at[pl.ds(b * _SRC + s0, _SCHUNK)], ubuf)
        pltpu.async_copy(ubuf, out_hbm.at[idxbuf.at[0]], sem)
        pltpu.make_async_copy(ubuf, out_hbm.at[idxbuf.at[0]], sem).wait()

    return _finalize_kernel


def kernel(logits, extended_vocab_ids, src_subtokens, src_padding,
           tgt_subtokens, len_vocab, max_len_extended_vocab,
           Wq, Wk, lin_w, lin_b):
    ids3 = extended_vocab_ids.reshape(_B, 1, _SRC)
    lw = lin_w.reshape(1, _D)
    lb = lin_b.reshape(1, 1)
    logits_t = jnp.swapaxes(logits, 1, 2)

    g = _make_gather_kernel()(
        extended_vocab_ids, logits_t.reshape(_B * _LV, _TGT))
    a2t, scal, idc = _attn_call(
        ids3, tgt_subtokens, src_subtokens, Wq, Wk, lw, lb)
    out_t, u = _combine_call(
        logits_t, scal, g.reshape(_B, _SRC, _TGT), a2t, idc)

    ref = jax.new_ref(out_t.reshape(_B * _V, _TGT))
    _make_finalize_kernel()(
        extended_vocab_ids, u.reshape(_B * _SRC, _TGT), ref)
    out_fin = jax.freeze(ref)
    return jnp.swapaxes(out_fin.reshape(_B, _V, _TGT), 1, 2)


# speedup vs baseline: 7.6843x; 1.0674x over previous
import functools
import math

import jax
import jax.numpy as jnp
from jax import lax
from jax.experimental import pallas as pl
from jax.experimental.pallas import tpu as pltpu
from jax.experimental.pallas import tpu_sc as plsc
import numpy as np

_B = 8
_SRC = 512
_TGT = 256
_D = 256
_H = 8
_LV = 8000
_MEXT = 512
_V = _LV + _MEXT
_DH = _D // _H

_EPS = float(np.finfo(np.float32).eps)
_LOG_EPS = float(np.log(np.finfo(np.float32).eps))
_FMIN = float(np.finfo(np.float32).min)

_VB = 2176
_NVB = -(-_V // _VB)
_NXB = -(-_LV // _VB)
_XW = _NXB * _VB

_WPB = 4
_SCHUNK = _SRC // _WPB


@functools.cache
def _make_gather_kernel():
    @functools.partial(
        pl.kernel,
        out_type=jax.ShapeDtypeStruct((_B * _SRC, _TGT), jnp.float32),
        mesh=plsc.VectorSubcoreMesh(core_axis_name="c", subcore_axis_name="s"),
        scratch_types=[
            pltpu.VMEM((_SCHUNK, _TGT), jnp.float32),
            pltpu.VMEM((1, _SCHUNK), jnp.int32),
            pltpu.VMEM((1, _SCHUNK), jnp.int32),
            pltpu.SemaphoreType.DMA,
        ],
        compiler_params=pltpu.CompilerParams(needs_layout_passes=False),
    )
    def _gather_kernel(ids_hbm, xt_hbm, g_hbm, gbuf, idsbuf, idxbuf, sem):
        core = lax.axis_index("c")
        sub = lax.axis_index("s")
        b = core * 4 + sub // _WPB
        w4 = sub % _WPB
        s0 = w4 * _SCHUNK

        pltpu.sync_copy(ids_hbm.at[b, pl.ds(s0, _SCHUNK)], idsbuf.at[0])
        for c in range(_SCHUNK // 16):
            iv = idsbuf[0, pl.ds(c * 16, 16)]
            idxbuf[0, pl.ds(c * 16, 16)] = (
                jnp.minimum(iv, _LV - 1) + b * _LV)
        pltpu.async_copy(xt_hbm.at[idxbuf.at[0]], gbuf, sem)
        pltpu.make_async_copy(xt_hbm.at[idxbuf.at[0]], gbuf, sem).wait()
        pltpu.sync_copy(gbuf, g_hbm.at[pl.ds(b * _SRC + s0, _SCHUNK)])

    return _gather_kernel


def _attn_body(ids_ref, tgt_ref, src_ref, wq_ref, wk_ref, lw_ref, lb_ref,
               a2t_ref, scal_ref, idc_ref):
    tgt = tgt_ref[0]
    src = src_ref[0]
    lw = lw_ref[...]

    z = lax.dot_general(lw, tgt, (((1,), (1,)), ((), ())),
                        preferred_element_type=jnp.float32)
    cp = jax.nn.sigmoid(z + lb_ref[...])
    scal_ref[0, 0:1, :] = jnp.log(cp)
    scal_ref[0, 1:2, :] = jnp.log(1.0 - cp)

    q = jnp.dot(tgt, wq_ref[...], preferred_element_type=jnp.float32)
    k = jnp.dot(src, wk_ref[...], preferred_element_type=jnp.float32)
    inv = jnp.float32(1.0 / math.sqrt(_DH))
    acc = jnp.zeros((_TGT, _SRC), jnp.float32)
    for h in range(_H):
        qh = q[:, h * _DH:(h + 1) * _DH]
        kh = k[:, h * _DH:(h + 1) * _DH]
        sc = lax.dot_general(qh, kh, (((1,), (1,)), ((), ())),
                             preferred_element_type=jnp.float32) * inv
        m = jnp.max(sc, axis=1, keepdims=True)
        e = jnp.exp(sc - m)
        acc = acc + e * (1.0 / jnp.sum(e, axis=1, keepdims=True))
    attn = acc * jnp.float32(1.0 / _H)

    idf = ids_ref[0].astype(jnp.float32)
    row = jnp.broadcast_to(idf, (_SRC, _SRC))
    col = row.T
    idc_ref[0] = col[:, 0:128]
    eq = (row == col).astype(jnp.float32)
    a2t_ref[0] = lax.dot_general(eq, attn, (((1,), (1,)), ((), ())),
                                 preferred_element_type=jnp.float32)


def _attn_call(ids3, tgt, src, wq, wk, lw, lb):
    return pl.pallas_call(
        _attn_body,
        grid=(_B,),
        in_specs=[
            pl.BlockSpec((1, 1, _SRC), lambda b: (b, 0, 0)),
            pl.BlockSpec((1, _TGT, _D), lambda b: (b, 0, 0)),
            pl.BlockSpec((1, _SRC, _D), lambda b: (b, 0, 0)),
            pl.BlockSpec((_D, _D), lambda b: (0, 0)),
            pl.BlockSpec((_D, _D), lambda b: (0, 0)),
            pl.BlockSpec((1, _D), lambda b: (0, 0)),
            pl.BlockSpec((1, 1), lambda b: (0, 0)),
        ],
        out_specs=[
            pl.BlockSpec((1, _SRC, _TGT), lambda b: (b, 0, 0)),
            pl.BlockSpec((1, 8, _TGT), lambda b: (b, 0, 0)),
            pl.BlockSpec((1, _SRC, 128), lambda b: (b, 0, 0)),
        ],
        out_shape=[
            jax.ShapeDtypeStruct((_B, _SRC, _TGT), jnp.float32),
            jax.ShapeDtypeStruct((_B, 8, _TGT), jnp.float32),
            jax.ShapeDtypeStruct((_B, _SRC, 128), jnp.float32),
        ],
        compiler_params=pltpu.CompilerParams(
            dimension_semantics=("arbitrary",)),
    )(ids3, tgt, src, wq, wk, lw, lb)


def _combine_body(x_ref, scal_ref, g_ref, a2t_ref, idc_ref,
                  out_ref, u_ref, lse_ref):
    vb = pl.program_id(1)
    logcp = scal_ref[0, 0:1, :]
    log1m = scal_ref[0, 1:2, :]

    @pl.when(vb == 0)
    def _():
        m = jnp.full((1, _TGT), _FMIN, jnp.float32)
        for k in range(_NXB):
            xk = x_ref[0, k * _VB:(k + 1) * _VB, :]
            if (k + 1) * _VB > _LV:
                rows = lax.broadcasted_iota(jnp.int32, (_VB, _TGT), 0)
                xk = jnp.where(rows < _LV - k * _VB, xk, _FMIN)
            m = jnp.maximum(m, jnp.max(xk, axis=0, keepdims=True))
        s = jnp.zeros((1, _TGT), jnp.float32)
        for k in range(_NXB):
            xk = x_ref[0, k * _VB:(k + 1) * _VB, :]
            if (k + 1) * _VB > _LV:
                rows = lax.broadcasted_iota(jnp.int32, (_VB, _TGT), 0)
                xk = jnp.where(rows < _LV - k * _VB, xk, _FMIN)
            s = s + jnp.sum(jnp.exp(xk - m), axis=0, keepdims=True)
        lse = m + jnp.log(s)
        lse_ref[...] = jnp.broadcast_to(lse, (8, _TGT))

        gd = jnp.where(idc_ref[0, :, 0:1] < float(_LV),
                       g_ref[0] - lse, -jnp.inf)
        p0u = gd + log1m
        p0u = jnp.where(p0u == -jnp.inf, _FMIN, p0u)
        a2 = a2t_ref[0]
        p1u = jnp.log(jnp.where(a2 == 0.0, _EPS, a2)) + logcp
        p1u = jnp.where(p1u == -jnp.inf, _FMIN, p1u)
        mu = jnp.maximum(p0u, p1u)
        u_ref[0] = mu + jnp.log1p(jnp.exp(-jnp.abs(p0u - p1u)))

    lse = lse_ref[0:1, :]
    c0 = jnp.maximum(log1m - lse, _FMIN)
    p1 = jnp.maximum(logcp + _LOG_EPS, _FMIN)
    x = x_ref[0, pl.ds(vb * _VB, _VB), :]
    vidx = lax.broadcasted_iota(jnp.int32, (_VB, _TGT), 0) + vb * _VB
    p0 = jnp.where(vidx < _LV, x + c0, _FMIN)
    mx = jnp.maximum(p0, p1)
    out_ref[0] = mx + jnp.log1p(jnp.exp(-jnp.abs(p0 - p1)))


def _combine_call(logits_t, scal, g, a2t, idc):
    return pl.pallas_call(
        _combine_body,
        grid=(_B, _NVB),
        in_specs=[
            pl.BlockSpec((1, _XW, _TGT), lambda b, v: (b, 0, 0)),
            pl.BlockSpec((1, 8, _TGT), lambda b, v: (b, 0, 0)),
            pl.BlockSpec((1, _SRC, _TGT), lambda b, v: (b, 0, 0)),
            pl.BlockSpec((1, _SRC, _TGT), lambda b, v: (b, 0, 0)),
            pl.BlockSpec((1, _SRC, 128), lambda b, v: (b, 0, 0)),
        ],
        out_specs=[
            pl.BlockSpec((1, _VB, _TGT), lambda b, v: (b, v, 0)),
            pl.BlockSpec((1, _SRC, _TGT), lambda b, v: (b, 0, 0)),
        ],
        out_shape=[
            jax.ShapeDtypeStruct((_B, _V, _TGT), jnp.float32),
            jax.ShapeDtypeStruct((_B, _SRC, _TGT), jnp.float32),
        ],
        scratch_shapes=[pltpu.VMEM((8, _TGT), jnp.float32)],
        compiler_params=pltpu.CompilerParams(
            dimension_semantics=("arbitrary", "arbitrary")),
    )(logits_t, scal, g, a2t, idc)


@functools.cache
def _make_finalize_kernel():
    @functools.partial(
        pl.kernel,
        out_type=(),
        mesh=plsc.VectorSubcoreMesh(core_axis_name="c", subcore_axis_name="s"),
        scratch_types=[
            pltpu.VMEM((_SCHUNK, _TGT), jnp.float32),
            pltpu.VMEM((1, _SCHUNK), jnp.int32),
            pltpu.VMEM((1, _SCHUNK), jnp.int32),
            pltpu.SemaphoreType.DMA,
        ],
        compiler_params=pltpu.CompilerParams(needs_layout_passes=False),
    )
    def _finalize_kernel(ids_hbm, u_hbm, out_hbm, ubuf, idsbuf, idxbuf, sem):
        core = lax.axis_index("c")
        sub = lax.axis_index("s")
        b = core * 4 + sub // _WPB
        w4 = sub % _WPB
        s0 = w4 * _SCHUNK

        pltpu.sync_copy(ids_hbm.at[b, pl.ds(s0, _SCHUNK)], idsbuf.at[0])
        for c in range(_SCHUNK // 16):
            idxbuf[0, pl.ds(c * 16, 16)] = (
                idsbuf[0, pl.ds(c * 16, 16)] + b * _V)
        pltpu.sync_copy(u_hbm.at[pl.ds(b * _SRC + s0, _SCHUNK)], ubuf)
        pltpu.async_copy(ubuf, out_hbm.at[idxbuf.at[0]], sem)
        pltpu.make_async_copy(ubuf, out_hbm.at[idxbuf.at[0]], sem).wait()

    return _finalize_kernel


def kernel(logits, extended_vocab_ids, src_subtokens, src_padding,
           tgt_subtokens, len_vocab, max_len_extended_vocab,
           Wq, Wk, lin_w, lin_b):
    ids3 = extended_vocab_ids.reshape(_B, 1, _SRC)
    lw = lin_w.reshape(1, _D)
    lb = lin_b.reshape(1, 1)
    logits_t = jnp.swapaxes(logits, 1, 2)

    g = _make_gather_kernel()(
        extended_vocab_ids, logits_t.reshape(_B * _LV, _TGT))
    a2t, scal, idc = _attn_call(
        ids3, tgt_subtokens, src_subtokens, Wq, Wk, lw, lb)
    out_t, u = _combine_call(
        logits_t, scal, g.reshape(_B, _SRC, _TGT), a2t, idc)

    ref = jax.new_ref(out_t.reshape(_B * _V, _TGT))
    _make_finalize_kernel()(
        extended_vocab_ids, u.reshape(_B * _SRC, _TGT), ref)
    out_fin = jax.freeze(ref)
    return jnp.swapaxes(out_fin.reshape(_B, _V, _TGT), 1, 2)


# speedup vs baseline: 7.6855x; 1.0002x over previous
import functools
import math

import jax
import jax.numpy as jnp
from jax import lax
from jax.experimental import pallas as pl
from jax.experimental.pallas import tpu as pltpu
from jax.experimental.pallas import tpu_sc as plsc
import numpy as np

_B = 8
_SRC = 512
_TGT = 256
_D = 256
_H = 8
_LV = 8000
_MEXT = 512
_V = _LV + _MEXT
_DH = _D // _H

_EPS = float(np.finfo(np.float32).eps)
_LOG_EPS = float(np.log(np.finfo(np.float32).eps))
_FMIN = float(np.finfo(np.float32).min)

_VB = 2176
_NVB = -(-_V // _VB)
_NXB = -(-_LV // _VB)
_XW = _NXB * _VB

_WPB = 4
_SCHUNK = _SRC // _WPB


@functools.cache
def _make_gather_kernel():
    @functools.partial(
        pl.kernel,
        out_type=jax.ShapeDtypeStruct((_B * _SRC, _TGT), jnp.float32),
        mesh=plsc.VectorSubcoreMesh(core_axis_name="c", subcore_axis_name="s"),
        scratch_types=[
            pltpu.VMEM((_SCHUNK, _TGT), jnp.float32),
            pltpu.VMEM((1, _SCHUNK), jnp.int32),
            pltpu.VMEM((1, _SCHUNK), jnp.int32),
            pltpu.SemaphoreType.DMA,
        ],
        compiler_params=pltpu.CompilerParams(needs_layout_passes=False),
    )
    def _gather_kernel(ids_hbm, xt_hbm, g_hbm, gbuf, idsbuf, idxbuf, sem):
        core = lax.axis_index("c")
        sub = lax.axis_index("s")
        b = core * 4 + sub // _WPB
        w4 = sub % _WPB
        s0 = w4 * _SCHUNK

        pltpu.sync_copy(ids_hbm.at[b, pl.ds(s0, _SCHUNK)], idsbuf.at[0])
        for c in range(_SCHUNK // 16):
            iv = idsbuf[0, pl.ds(c * 16, 16)]
            idxbuf[0, pl.ds(c * 16, 16)] = (
                jnp.minimum(iv, _LV - 1) + b * _LV)
        pltpu.async_copy(xt_hbm.at[idxbuf.at[0]], gbuf, sem)
        pltpu.make_async_copy(xt_hbm.at[idxbuf.at[0]], gbuf, sem).wait()
        pltpu.sync_copy(gbuf, g_hbm.at[pl.ds(b * _SRC + s0, _SCHUNK)])

    return _gather_kernel


def _attn_body(ids_ref, tgt_ref, src_ref, wq_ref, wk_ref, lw_ref, lb_ref,
               a2t_ref, scal_ref, idc_ref):
    tgt = tgt_ref[0]
    src = src_ref[0]
    lw = lw_ref[...]

    z = lax.dot_general(lw, tgt, (((1,), (1,)), ((), ())),
                        preferred_element_type=jnp.float32)
    cp = jax.nn.sigmoid(z + lb_ref[...])
    scal_ref[0, 0:1, :] = jnp.log(cp)
    scal_ref[0, 1:2, :] = jnp.log(1.0 - cp)

    q = jnp.dot(tgt, wq_ref[...], preferred_element_type=jnp.float32)
    k = jnp.dot(src, wk_ref[...], preferred_element_type=jnp.float32)
    inv = jnp.float32(1.0 / math.sqrt(_DH))
    acc = jnp.zeros((_TGT, _SRC), jnp.float32)
    for h in range(_H):
        qh = q[:, h * _DH:(h + 1) * _DH]
        kh = k[:, h * _DH:(h + 1) * _DH]
        sc = lax.dot_general(qh, kh, (((1,), (1,)), ((), ())),
                             preferred_element_type=jnp.float32) * inv
        m = jnp.max(sc, axis=1, keepdims=True)
        e = jnp.exp(sc - m)
        acc = acc + e * (1.0 / jnp.sum(e, axis=1, keepdims=True))
    attn = acc * jnp.float32(1.0 / _H)

    idf = ids_ref[0].astype(jnp.float32)
    row = jnp.broadcast_to(idf, (_SRC, _SRC))
    col = row.T
    idc_ref[0] = col[:, 0:128]
    eq = (row == col).astype(jnp.float32)
    a2t_ref[0] = lax.dot_general(eq, attn, (((1,), (1,)), ((), ())),
                                 preferred_element_type=jnp.float32)


def _attn_call(ids3, tgt, src, wq, wk, lw, lb):
    return pl.pallas_call(
        _attn_body,
        grid=(_B,),
        in_specs=[
            pl.BlockSpec((1, 1, _SRC), lambda b: (b, 0, 0)),
            pl.BlockSpec((1, _TGT, _D), lambda b: (b, 0, 0)),
            pl.BlockSpec((1, _SRC, _D), lambda b: (b, 0, 0)),
            pl.BlockSpec((_D, _D), lambda b: (0, 0)),
            pl.BlockSpec((_D, _D), lambda b: (0, 0)),
            pl.BlockSpec((1, _D), lambda b: (0, 0)),
            pl.BlockSpec((1, 1), lambda b: (0, 0)),
        ],
        out_specs=[
            pl.BlockSpec((1, _SRC, _TGT), lambda b: (b, 0, 0)),
            pl.BlockSpec((1, 8, _TGT), lambda b: (b, 0, 0)),
            pl.BlockSpec((1, _SRC, 128), lambda b: (b, 0, 0)),
        ],
        out_shape=[
            jax.ShapeDtypeStruct((_B, _SRC, _TGT), jnp.float32),
            jax.ShapeDtypeStruct((_B, 8, _TGT), jnp.float32),
            jax.ShapeDtypeStruct((_B, _SRC, 128), jnp.float32),
        ],
        compiler_params=pltpu.CompilerParams(
            dimension_semantics=("arbitrary",)),
    )(ids3, tgt, src, wq, wk, lw, lb)


def _combine_body(x_ref, scal_ref, g_ref, a2t_ref, idc_ref,
                  out_ref, u_ref, lse_ref):
    vb = pl.program_id(1)
    logcp = scal_ref[0, 0:1, :]
    log1m = scal_ref[0, 1:2, :]

    @pl.when(vb == 0)
    def _():
        m = jnp.full((1, _TGT), _FMIN, jnp.float32)
        s = jnp.zeros((1, _TGT), jnp.float32)
        for k in range(_NXB):
            xk = x_ref[0, k * _VB:(k + 1) * _VB, :]
            if (k + 1) * _VB > _LV:
                rows = lax.broadcasted_iota(jnp.int32, (_VB, _TGT), 0)
                xk = jnp.where(rows < _LV - k * _VB, xk, _FMIN)
            m2 = jnp.maximum(m, jnp.max(xk, axis=0, keepdims=True))
            s = s * jnp.exp(m - m2) + jnp.sum(
                jnp.exp(xk - m2), axis=0, keepdims=True)
            m = m2
        lse = m + jnp.log(s)
        lse_ref[...] = jnp.broadcast_to(lse, (8, _TGT))

        gd = jnp.where(idc_ref[0, :, 0:1] < float(_LV),
                       g_ref[0] - lse, -jnp.inf)
        p0u = gd + log1m
        p0u = jnp.where(p0u == -jnp.inf, _FMIN, p0u)
        a2 = a2t_ref[0]
        p1u = jnp.log(jnp.where(a2 == 0.0, _EPS, a2)) + logcp
        p1u = jnp.where(p1u == -jnp.inf, _FMIN, p1u)
        mu = jnp.maximum(p0u, p1u)
        u_ref[0] = mu + jnp.log1p(jnp.exp(-jnp.abs(p0u - p1u)))

    lse = lse_ref[0:1, :]
    c0 = jnp.maximum(log1m - lse, _FMIN)
    p1 = jnp.maximum(logcp + _LOG_EPS, _FMIN)
    x = x_ref[0, pl.ds(vb * _VB, _VB), :]
    vidx = lax.broadcasted_iota(jnp.int32, (_VB, _TGT), 0) + vb * _VB
    p0 = jnp.where(vidx < _LV, x + c0, _FMIN)
    mx = jnp.maximum(p0, p1)
    out_ref[0] = mx + jnp.log1p(jnp.exp(-jnp.abs(p0 - p1)))


def _combine_call(logits_t, scal, g, a2t, idc):
    return pl.pallas_call(
        _combine_body,
        grid=(_B, _NVB),
        in_specs=[
            pl.BlockSpec((1, _XW, _TGT), lambda b, v: (b, 0, 0)),
            pl.BlockSpec((1, 8, _TGT), lambda b, v: (b, 0, 0)),
            pl.BlockSpec((1, _SRC, _TGT), lambda b, v: (b, 0, 0)),
            pl.BlockSpec((1, _SRC, _TGT), lambda b, v: (b, 0, 0)),
            pl.BlockSpec((1, _SRC, 128), lambda b, v: (b, 0, 0)),
        ],
        out_specs=[
            pl.BlockSpec((1, _VB, _TGT), lambda b, v: (b, v, 0)),
            pl.BlockSpec((1, _SRC, _TGT), lambda b, v: (b, 0, 0)),
        ],
        out_shape=[
            jax.ShapeDtypeStruct((_B, _V, _TGT), jnp.float32),
            jax.ShapeDtypeStruct((_B, _SRC, _TGT), jnp.float32),
        ],
        scratch_shapes=[pltpu.VMEM((8, _TGT), jnp.float32)],
        compiler_params=pltpu.CompilerParams(
            dimension_semantics=("arbitrary", "arbitrary")),
    )(logits_t, scal, g, a2t, idc)


@functools.cache
def _make_finalize_kernel():
    @functools.partial(
        pl.kernel,
        out_type=(),
        mesh=plsc.VectorSubcoreMesh(core_axis_name="c", subcore_axis_name="s"),
        scratch_types=[
            pltpu.VMEM((_SCHUNK, _TGT), jnp.float32),
            pltpu.VMEM((1, _SCHUNK), jnp.int32),
            pltpu.VMEM((1, _SCHUNK), jnp.int32),
            pltpu.SemaphoreType.DMA,
        ],
        compiler_params=pltpu.CompilerParams(needs_layout_passes=False),
    )
    def _finalize_kernel(ids_hbm, u_hbm, out_hbm, ubuf, idsbuf, idxbuf, sem):
        core = lax.axis_index("c")
        sub = lax.axis_index("s")
        b = core * 4 + sub // _WPB
        w4 = sub % _WPB
        s0 = w4 * _SCHUNK

        pltpu.sync_copy(ids_hbm.at[b, pl.ds(s0, _SCHUNK)], idsbuf.at[0])
        for c in range(_SCHUNK // 16):
            idxbuf[0, pl.ds(c * 16, 16)] = (
                idsbuf[0, pl.ds(c * 16, 16)] + b * _V)
        pltpu.sync_copy(u_hbm.at[pl.ds(b * _SRC + s0, _SCHUNK)], ubuf)
        pltpu.async_copy(ubuf, out_hbm.at[idxbuf.at[0]], sem)
        pltpu.make_async_copy(ubuf, out_hbm.at[idxbuf.at[0]], sem).wait()

    return _finalize_kernel


def kernel(logits, extended_vocab_ids, src_subtokens, src_padding,
           tgt_subtokens, len_vocab, max_len_extended_vocab,
           Wq, Wk, lin_w, lin_b):
    ids3 = extended_vocab_ids.reshape(_B, 1, _SRC)
    lw = lin_w.reshape(1, _D)
    lb = lin_b.reshape(1, 1)
    logits_t = jnp.swapaxes(logits, 1, 2)

    g = _make_gather_kernel()(
        extended_vocab_ids, logits_t.reshape(_B * _LV, _TGT))
    a2t, scal, idc = _attn_call(
        ids3, tgt_subtokens, src_subtokens, Wq, Wk, lw, lb)
    out_t, u = _combine_call(
        logits_t, scal, g.reshape(_B, _SRC, _TGT), a2t, idc)

    ref = jax.new_ref(out_t.reshape(_B * _V, _TGT))
    _make_finalize_kernel()(
        extended_vocab_ids, u.reshape(_B * _SRC, _TGT), ref)
    out_fin = jax.freeze(ref)
    return jnp.swapaxes(out_fin.reshape(_B, _V, _TGT), 1, 2)
